# baseline (speedup 1.0000x reference)
import functools

import jax
import jax.numpy as jnp
from jax import lax
from jax.experimental import pallas as pl
from jax.experimental.pallas import tpu as pltpu
from jax.experimental.pallas import tpu_sc as plsc

B, T, D = 1, 2048, 768
H, E, K, R = 1024, 8, 2, 5
LBW = 0.01

BT = 512
NT = T // BT
BG = 384
GP = 18 * BG
NB = GP // BG

NW = 32
TPW = T // NW
CH = 16


def _router_kernel(x_ref, regime_ref, gamma_ref, beta_ref,
                   wr1_ref, br1_ref, wr2_ref, br2_ref,
                   xn_ref, w0_ref, w1_ref, s0_ref, s1_ref,
                   blke_ref, nblk_ref, aux_ref,
                   idx_scr, win_scr, cnt_scr, aux_acc):
    p = pl.program_id(0)
    t = pl.program_id(1)

    @pl.when(p == 0)
    def _phase0():
        xblk = x_ref[...]
        mean = jnp.mean(xblk, axis=1, keepdims=True)
        xc = xblk - mean
        var = jnp.mean(xc * xc, axis=1, keepdims=True)
        xn = xc * jax.lax.rsqrt(var + 1e-5) * gamma_ref[...] + beta_ref[...]
        xn_ref[...] = xn
        rc = jnp.dot(regime_ref[...], wr1_ref[D:D + R, :],
                     preferred_element_type=jnp.float32)
        hpre = (jnp.dot(xn, wr1_ref[0:D, :],
                        preferred_element_type=jnp.float32)
                + rc + br1_ref[...])
        hrt = hpre * jax.nn.sigmoid(hpre)
        logits = (jnp.dot(hrt, wr2_ref[...],
                          preferred_element_type=jnp.float32)
                  + br2_ref[...])
        ecols = jax.lax.broadcasted_iota(jnp.int32, (BT, E), 1)
        m1 = jnp.max(logits, axis=1, keepdims=True)
        i1 = jnp.min(jnp.where(logits == m1, ecols, E), axis=1, keepdims=True)
        masked = jnp.where(ecols == i1, -jnp.inf, logits)
        m2 = jnp.max(masked, axis=1, keepdims=True)
        i2 = jnp.min(jnp.where(masked == m2, ecols, E), axis=1, keepdims=True)
        w_first = 1.0 / (1.0 + jnp.exp(m2 - m1))
        idx_scr[pl.ds(t * BT, BT), :] = jnp.concatenate([i1, i2], axis=1)
        w0_ref[...] = jnp.broadcast_to(w_first, (BT, 16))
        w1_ref[...] = jnp.broadcast_to(1.0 - w_first, (BT, 16))
        oh1 = (ecols == i1).astype(jnp.float32)
        oh2 = (ecols == i2).astype(jnp.float32)
        oh_both = oh1 + oh2
        rr = jax.lax.broadcasted_iota(jnp.int32, (BT, BT), 0)
        cc = jax.lax.broadcasted_iota(jnp.int32, (BT, BT), 1)
        tril = (rr > cc).astype(jnp.float32)
        before = jax.lax.dot_general(
            tril, oh_both, (((1,), (0,)), ((), ())),
            preferred_element_type=jnp.float32)
        win1 = jnp.sum(before * oh1, axis=1, keepdims=True)
        win2 = jnp.sum(before * oh2, axis=1, keepdims=True)
        win_scr[pl.ds(t * BT, BT), :] = jnp.concatenate([win1, win2], axis=1)
        cnt_scr[pl.ds(t, 1), :] = jnp.sum(oh_both, axis=0, keepdims=True)
        prob = jnp.exp(logits - m1)
        prob = prob / jnp.sum(prob, axis=1, keepdims=True)
        pa = jnp.sum(prob, axis=0, keepdims=True) / T
        ma = jnp.sum(oh1, axis=0, keepdims=True) / T

        @pl.when(t == 0)
        def _():
            aux_acc[0:1, 0:E] = pa
            aux_acc[1:2, 0:E] = ma

        @pl.when(t > 0)
        def _():
            aux_acc[0:1, 0:E] += pa
            aux_acc[1:2, 0:E] += ma

        @pl.when(t == NT - 1)
        def _():
            aux_ref[...] = (LBW * E) * jnp.sum(
                aux_acc[0:1, 0:E] * aux_acc[1:2, 0:E], axis=1, keepdims=True)

    @pl.when(p == 1)
    def _phase1():
        cnt_all = jnp.sum(cnt_scr[...], axis=0, keepdims=True)
        pc = jnp.ceil(cnt_all * (1.0 / BG)) * BG
        r8 = jax.lax.broadcasted_iota(jnp.int32, (E, E), 0)
        c8 = jax.lax.broadcasted_iota(jnp.int32, (E, E), 1)
        upper = (r8 < c8).astype(jnp.float32)
        seg_start = jnp.dot(pc, upper,
                            preferred_element_type=jnp.float32)
        rows_nt = jax.lax.broadcasted_iota(jnp.int32, (NT, E), 0)
        before_blk = jnp.sum(jnp.where(rows_nt < t, cnt_scr[...], 0.0),
                             axis=0, keepdims=True)
        gbase = seg_start + before_blk
        idx = idx_scr[pl.ds(t * BT, BT), :]
        win = win_scr[pl.ds(t * BT, BT), :]
        ecols = jax.lax.broadcasted_iota(jnp.int32, (BT, E), 1)
        oh1 = (ecols == idx[:, 0:1]).astype(jnp.float32)
        oh2 = (ecols == idx[:, 1:2]).astype(jnp.float32)
        g1 = jnp.sum(oh1 * gbase, axis=1, keepdims=True)
        g2 = jnp.sum(oh2 * gbase, axis=1, keepdims=True)
        s0_ref[...] = (g1 + win[:, 0:1]).astype(jnp.int32)
        s1_ref[...] = (g2 + win[:, 1:2]).astype(jnp.int32)

        @pl.when(t == 0)
        def _():
            nblk_ref[...] = (jnp.sum(pc, axis=1, keepdims=True)
                             * (1.0 / BG)).astype(jnp.int32)
            biota = jax.lax.broadcasted_iota(jnp.int32, (1, NB), 1)
            acc = jnp.zeros((1, NB), jnp.int32)
            bstart = (seg_start * (1.0 / BG)).astype(jnp.int32)
            for ee in range(E):
                acc += (biota >= bstart[0:1, ee:ee + 1]).astype(jnp.int32)
            blke_ref[...] = acc - 1


def _router(x2d, regime, gamma, beta, wr1, br1, wr2, br2):
    def _p0map(p, t):
        return (jnp.where(p == 0, t, NT), 0)

    def _p1map(p, t):
        return (t, 0)

    return pl.pallas_call(
        _router_kernel,
        grid=(2, NT),
        in_specs=[
            pl.BlockSpec((BT, D), lambda p, t: (t, 0)),
            pl.BlockSpec((B, R), lambda p, t: (0, 0)),
            pl.BlockSpec((1, D), lambda p, t: (0, 0)),
            pl.BlockSpec((1, D), lambda p, t: (0, 0)),
            pl.BlockSpec((D + R, D), lambda p, t: (0, 0)),
            pl.BlockSpec((1, D), lambda p, t: (0, 0)),
            pl.BlockSpec((D, E), lambda p, t: (0, 0)),
            pl.BlockSpec((1, E), lambda p, t: (0, 0)),
        ],
        out_specs=[
            pl.BlockSpec((BT, D), _p0map),
            pl.BlockSpec((BT, 16), _p0map),
            pl.BlockSpec((BT, 16), _p0map),
            pl.BlockSpec((BT, 1), _p1map),
            pl.BlockSpec((BT, 1), _p1map),
            pl.BlockSpec((1, NB), lambda p, t: (0, 0)),
            pl.BlockSpec((1, 1), lambda p, t: (0, 0)),
            pl.BlockSpec((1, 1), lambda p, t: (0, 0)),
        ],
        out_shape=[
            jax.ShapeDtypeStruct((T + BT, D), jnp.float32),
            jax.ShapeDtypeStruct((T + BT, 16), jnp.float32),
            jax.ShapeDtypeStruct((T + BT, 16), jnp.float32),
            jax.ShapeDtypeStruct((T, 1), jnp.int32),
            jax.ShapeDtypeStruct((T, 1), jnp.int32),
            jax.ShapeDtypeStruct((1, NB), jnp.int32),
            jax.ShapeDtypeStruct((1, 1), jnp.int32),
            jax.ShapeDtypeStruct((1, 1), jnp.float32),
        ],
        scratch_shapes=[
            pltpu.VMEM((T, K), jnp.int32),
            pltpu.VMEM((T, K), jnp.float32),
            pltpu.VMEM((NT, E), jnp.float32),
            pltpu.VMEM((8, 128), jnp.float32),
        ],
    )(x2d, regime, gamma, beta, wr1, br1, wr2, br2)


_SC_MESH = plsc.VectorSubcoreMesh(core_axis_name="c", subcore_axis_name="s")


@functools.partial(
    pl.kernel, mesh=_SC_MESH,
    out_type=jax.ShapeDtypeStruct((GP, D), jnp.float32),
    scratch_types=[
        pltpu.VMEM((TPW, D), jnp.float32),
        pltpu.VMEM((TPW,), jnp.int32),
        pltpu.VMEM((TPW,), jnp.int32),
        pltpu.SemaphoreType.DMA,
        pltpu.SemaphoreType.DMA,
        pltpu.SemaphoreType.DMA,
    ],
)
def _sc_scatter(xn_hbm, s0_hbm, s1_hbm, xg_hbm, rows_v, i0_v, i1_v,
                sr, sa, sb):
    w = lax.axis_index("s") * 2 + lax.axis_index("c")
    cr = pltpu.async_copy(xn_hbm.at[pl.ds(w * TPW, TPW)], rows_v, sr)
    c0 = pltpu.async_copy(s0_hbm.at[pl.ds(w * TPW, TPW)], i0_v, sa)
    c1 = pltpu.async_copy(s1_hbm.at[pl.ds(w * TPW, TPW)], i1_v, sb)
    cr.wait()
    c0.wait()
    c1.wait()
    g0 = pltpu.async_copy(rows_v, xg_hbm.at[i0_v], sa)
    g1 = pltpu.async_copy(rows_v, xg_hbm.at[i1_v], sb)
    g0.wait()
    g1.wait()


_CSET = [
    pltpu.VMEM((CH, D), jnp.float32),
    pltpu.VMEM((CH, D), jnp.float32),
    pltpu.VMEM((CH, D), jnp.float32),
    pltpu.VMEM((CH,), jnp.int32),
    pltpu.VMEM((CH,), jnp.int32),
    pltpu.VMEM((CH, 16), jnp.float32),
    pltpu.VMEM((CH, 16), jnp.float32),
]


@functools.partial(
    pl.kernel, mesh=_SC_MESH,
    out_type=jax.ShapeDtypeStruct((T, D), jnp.float32),
    scratch_types=_CSET + _CSET + [
        pltpu.SemaphoreType.DMA,
        pltpu.SemaphoreType.DMA,
        pltpu.SemaphoreType.DMA,
        pltpu.SemaphoreType.DMA,
    ],
)
def _sc_combine(x_hbm, yg_hbm, s0_hbm, s1_hbm, wq0_hbm, wq1_hbm, out_hbm,
                *bufs):
    sets = (bufs[0:7], bufs[7:14])
    sld = bufs[14:16]
    sg = bufs[16:18]
    w = lax.axis_index("s") * 2 + lax.axis_index("c")
    nch = TPW // CH

    def fire_loads(c):
        xr, _, _, i0, i1, w0, w1 = sets[c % 2]
        sem = sld[c % 2]
        base = w * TPW + c * CH
        return [
            pltpu.async_copy(s0_hbm.at[pl.ds(base, CH)], i0, sem),
            pltpu.async_copy(s1_hbm.at[pl.ds(base, CH)], i1, sem),
            pltpu.async_copy(wq0_hbm.at[pl.ds(base, CH)], w0, sem),
            pltpu.async_copy(wq1_hbm.at[pl.ds(base, CH)], w1, sem),
            pltpu.async_copy(x_hbm.at[pl.ds(base, CH)], xr, sem),
        ]

    def fire_gathers(c):
        _, r0, r1, i0, i1, _, _ = sets[c % 2]
        sem = sg[c % 2]
        return [
            pltpu.async_copy(yg_hbm.at[i0], r0, sem),
            pltpu.async_copy(yg_hbm.at[i1], r1, sem),
        ]

    for h in fire_loads(0):
        h.wait()
    gathers = fire_gathers(0)
    for c in range(nch):
        xr, r0, r1, _, _, w0, w1 = sets[c % 2]
        loads_next = fire_loads(c + 1) if c + 1 < nch else None
        for h in gathers:
            h.wait()
        if loads_next is not None:
            for h in loads_next:
                h.wait()
            gathers = fire_gathers(c + 1)

        def body(r, carry):
            w0s = w0[r, :]
            w1s = w1[r, :]
            for c16 in range(D // 16):
                sl = pl.ds(c16 * 16, 16)
                xr[r, sl] = (xr[r, sl] + r0[r, sl] * w0s
                             + r1[r, sl] * w1s)
            return carry

        lax.fori_loop(0, CH, body, 0)
        pltpu.sync_copy(xr, out_hbm.at[pl.ds(w * TPW + c * CH, CH)])


def _expert_kernel(nblk_ref, blke_ref, xg_ref,
                   w1_ref, b1_ref, w2_ref, b2_ref, yg_ref):
    b = pl.program_id(0)

    @pl.when(b < nblk_ref[0])
    def _():
        h = (jax.lax.dot_general(
            xg_ref[...], w1_ref[0], (((1,), (0,)), ((), ())),
            precision=jax.lax.Precision.DEFAULT,
            preferred_element_type=jnp.float32) + b1_ref[0])
        h = h * jax.nn.sigmoid(h)
        yg_ref[...] = (jax.lax.dot_general(
            h, w2_ref[0], (((1,), (0,)), ((), ())),
            precision=jax.lax.Precision.DEFAULT,
            preferred_element_type=jnp.float32) + b2_ref[0])


def _experts(nblocks, blk_e, xg, w1b, b1r, w2b, b2r):
    def _rowmap(b, n, e):
        return (jnp.minimum(b, n[0] - 1), 0)

    def _emap3(b, n, e):
        return (e[jnp.minimum(b, n[0] - 1)], 0, 0)

    grid_spec = pltpu.PrefetchScalarGridSpec(
        num_scalar_prefetch=2,
        grid=(NB,),
        in_specs=[
            pl.BlockSpec((BG, D), _rowmap),
            pl.BlockSpec((1, D, H), _emap3),
            pl.BlockSpec((1, 1, H), _emap3),
            pl.BlockSpec((1, H, D), _emap3),
            pl.BlockSpec((1, 1, D), _emap3),
        ],
        out_specs=pl.BlockSpec((BG, D), lambda b, n, e: (b, 0)),
    )
    return pl.pallas_call(
        _expert_kernel,
        grid_spec=grid_spec,
        out_shape=jax.ShapeDtypeStruct((GP, D), jnp.float32),
    )(nblocks, blk_e, xg, w1b, b1r, w2b, b2r)


def kernel(x, regime, ln_gamma, ln_beta, W1, b1, W2, b2, Wr1, br1, Wr2, br2):
    x2d = x.reshape(T, D)
    xn, w0, w1, s0, s1, blke2, nblk2, aux = _router(
        x2d, regime, ln_gamma.reshape(1, D), ln_beta.reshape(1, D),
        Wr1, br1.reshape(1, D), Wr2, br2.reshape(1, E))

    s0f = s0.reshape(T)
    s1f = s1.reshape(T)
    xg = _sc_scatter(xn, s0f, s1f)
    yg = _experts(nblk2.reshape(1), blke2.reshape(NB), xg,
                  W1, b1.reshape(E, 1, H),
                  W2, b2.reshape(E, 1, D))
    out2d = _sc_combine(x2d, yg, s0f, s1f, w0, w1)
    return out2d.reshape(B, T, D), aux[0, 0]


# speedup vs baseline: 1.0581x; 1.0581x over previous
import functools

import jax
import jax.numpy as jnp
from jax import lax
from jax.experimental import pallas as pl
from jax.experimental.pallas import tpu as pltpu
from jax.experimental.pallas import tpu_sc as plsc

B, T, D = 1, 2048, 768
H, E, K, R = 1024, 8, 2, 5
LBW = 0.01

BT = 512
NT = T // BT
BG = 512
GP = T * K + E * BG
NB = GP // BG

NW = 32
TPW = T // NW
CH = 16


def _router_kernel(x_ref, regime_ref, gamma_ref, beta_ref,
                   wr1_ref, br1_ref, wr2_ref, br2_ref,
                   xn_ref, w0_ref, w1_ref, s0_ref, s1_ref,
                   blke_ref, nblk_ref, aux_ref,
                   idx_scr, win_scr, cnt_scr, aux_acc):
    p = pl.program_id(0)
    t = pl.program_id(1)

    @pl.when(p == 0)
    def _phase0():
        xblk = x_ref[...]
        mean = jnp.mean(xblk, axis=1, keepdims=True)
        xc = xblk - mean
        var = jnp.mean(xc * xc, axis=1, keepdims=True)
        xn = xc * jax.lax.rsqrt(var + 1e-5) * gamma_ref[...] + beta_ref[...]
        xn_ref[...] = xn
        rc = jnp.dot(regime_ref[...], wr1_ref[D:D + R, :],
                     preferred_element_type=jnp.float32)
        hpre = (jnp.dot(xn, wr1_ref[0:D, :],
                        preferred_element_type=jnp.float32)
                + rc + br1_ref[...])
        hrt = hpre * jax.nn.sigmoid(hpre)
        logits = (jnp.dot(hrt, wr2_ref[...],
                          preferred_element_type=jnp.float32)
                  + br2_ref[...])
        ecols = jax.lax.broadcasted_iota(jnp.int32, (BT, E), 1)
        m1 = jnp.max(logits, axis=1, keepdims=True)
        i1 = jnp.min(jnp.where(logits == m1, ecols, E), axis=1, keepdims=True)
        masked = jnp.where(ecols == i1, -jnp.inf, logits)
        m2 = jnp.max(masked, axis=1, keepdims=True)
        i2 = jnp.min(jnp.where(masked == m2, ecols, E), axis=1, keepdims=True)
        w_first = 1.0 / (1.0 + jnp.exp(m2 - m1))
        idx_scr[pl.ds(t * BT, BT), :] = jnp.concatenate([i1, i2], axis=1)
        w0_ref[...] = jnp.broadcast_to(w_first, (BT, 16))
        w1_ref[...] = jnp.broadcast_to(1.0 - w_first, (BT, 16))
        oh1 = (ecols == i1).astype(jnp.float32)
        oh2 = (ecols == i2).astype(jnp.float32)
        oh_both = oh1 + oh2
        rr = jax.lax.broadcasted_iota(jnp.int32, (BT, BT), 0)
        cc = jax.lax.broadcasted_iota(jnp.int32, (BT, BT), 1)
        tril = (rr > cc).astype(jnp.float32)
        before = jax.lax.dot_general(
            tril, oh_both, (((1,), (0,)), ((), ())),
            preferred_element_type=jnp.float32)
        win1 = jnp.sum(before * oh1, axis=1, keepdims=True)
        win2 = jnp.sum(before * oh2, axis=1, keepdims=True)
        win_scr[pl.ds(t * BT, BT), :] = jnp.concatenate([win1, win2], axis=1)
        cnt_scr[pl.ds(t, 1), :] = jnp.sum(oh_both, axis=0, keepdims=True)
        prob = jnp.exp(logits - m1)
        prob = prob / jnp.sum(prob, axis=1, keepdims=True)
        pa = jnp.sum(prob, axis=0, keepdims=True) / T
        ma = jnp.sum(oh1, axis=0, keepdims=True) / T

        @pl.when(t == 0)
        def _():
            aux_acc[0:1, 0:E] = pa
            aux_acc[1:2, 0:E] = ma

        @pl.when(t > 0)
        def _():
            aux_acc[0:1, 0:E] += pa
            aux_acc[1:2, 0:E] += ma

        @pl.when(t == NT - 1)
        def _():
            aux_ref[...] = (LBW * E) * jnp.sum(
                aux_acc[0:1, 0:E] * aux_acc[1:2, 0:E], axis=1, keepdims=True)

    @pl.when(p == 1)
    def _phase1():
        cnt_all = jnp.sum(cnt_scr[...], axis=0, keepdims=True)
        pc = jnp.ceil(cnt_all * (1.0 / BG)) * BG
        r8 = jax.lax.broadcasted_iota(jnp.int32, (E, E), 0)
        c8 = jax.lax.broadcasted_iota(jnp.int32, (E, E), 1)
        upper = (r8 < c8).astype(jnp.float32)
        seg_start = jnp.dot(pc, upper,
                            preferred_element_type=jnp.float32)
        rows_nt = jax.lax.broadcasted_iota(jnp.int32, (NT, E), 0)
        before_blk = jnp.sum(jnp.where(rows_nt < t, cnt_scr[...], 0.0),
                             axis=0, keepdims=True)
        gbase = seg_start + before_blk
        idx = idx_scr[pl.ds(t * BT, BT), :]
        win = win_scr[pl.ds(t * BT, BT), :]
        ecols = jax.lax.broadcasted_iota(jnp.int32, (BT, E), 1)
        oh1 = (ecols == idx[:, 0:1]).astype(jnp.float32)
        oh2 = (ecols == idx[:, 1:2]).astype(jnp.float32)
        g1 = jnp.sum(oh1 * gbase, axis=1, keepdims=True)
        g2 = jnp.sum(oh2 * gbase, axis=1, keepdims=True)
        s0_ref[...] = (g1 + win[:, 0:1]).astype(jnp.int32)
        s1_ref[...] = (g2 + win[:, 1:2]).astype(jnp.int32)

        @pl.when(t == 0)
        def _():
            nblk_ref[...] = (jnp.sum(pc, axis=1, keepdims=True)
                             * (1.0 / BG)).astype(jnp.int32)
            biota = jax.lax.broadcasted_iota(jnp.int32, (1, NB), 1)
            acc = jnp.zeros((1, NB), jnp.int32)
            bstart = (seg_start * (1.0 / BG)).astype(jnp.int32)
            for ee in range(E):
                acc += (biota >= bstart[0:1, ee:ee + 1]).astype(jnp.int32)
            blke_ref[...] = acc - 1


def _router(x2d, regime, gamma, beta, wr1, br1, wr2, br2):
    def _p0map(p, t):
        return (jnp.where(p == 0, t, NT), 0)

    def _p1map(p, t):
        return (t, 0)

    return pl.pallas_call(
        _router_kernel,
        grid=(2, NT),
        in_specs=[
            pl.BlockSpec((BT, D), lambda p, t: (t, 0)),
            pl.BlockSpec((B, R), lambda p, t: (0, 0)),
            pl.BlockSpec((1, D), lambda p, t: (0, 0)),
            pl.BlockSpec((1, D), lambda p, t: (0, 0)),
            pl.BlockSpec((D + R, D), lambda p, t: (0, 0)),
            pl.BlockSpec((1, D), lambda p, t: (0, 0)),
            pl.BlockSpec((D, E), lambda p, t: (0, 0)),
            pl.BlockSpec((1, E), lambda p, t: (0, 0)),
        ],
        out_specs=[
            pl.BlockSpec((BT, D), _p0map),
            pl.BlockSpec((BT, 16), _p0map),
            pl.BlockSpec((BT, 16), _p0map),
            pl.BlockSpec((BT, 1), _p1map),
            pl.BlockSpec((BT, 1), _p1map),
            pl.BlockSpec((1, NB), lambda p, t: (0, 0)),
            pl.BlockSpec((1, 1), lambda p, t: (0, 0)),
            pl.BlockSpec((1, 1), lambda p, t: (0, 0)),
        ],
        out_shape=[
            jax.ShapeDtypeStruct((T + BT, D), jnp.float32),
            jax.ShapeDtypeStruct((T + BT, 16), jnp.float32),
            jax.ShapeDtypeStruct((T + BT, 16), jnp.float32),
            jax.ShapeDtypeStruct((T, 1), jnp.int32),
            jax.ShapeDtypeStruct((T, 1), jnp.int32),
            jax.ShapeDtypeStruct((1, NB), jnp.int32),
            jax.ShapeDtypeStruct((1, 1), jnp.int32),
            jax.ShapeDtypeStruct((1, 1), jnp.float32),
        ],
        scratch_shapes=[
            pltpu.VMEM((T, K), jnp.int32),
            pltpu.VMEM((T, K), jnp.float32),
            pltpu.VMEM((NT, E), jnp.float32),
            pltpu.VMEM((8, 128), jnp.float32),
        ],
    )(x2d, regime, gamma, beta, wr1, br1, wr2, br2)


_SC_MESH = plsc.VectorSubcoreMesh(core_axis_name="c", subcore_axis_name="s")


@functools.partial(
    pl.kernel, mesh=_SC_MESH,
    out_type=jax.ShapeDtypeStruct((GP, D), jnp.float32),
    scratch_types=[
        pltpu.VMEM((TPW, D), jnp.float32),
        pltpu.VMEM((TPW,), jnp.int32),
        pltpu.VMEM((TPW,), jnp.int32),
        pltpu.SemaphoreType.DMA,
        pltpu.SemaphoreType.DMA,
        pltpu.SemaphoreType.DMA,
    ],
)
def _sc_scatter(xn_hbm, s0_hbm, s1_hbm, xg_hbm, rows_v, i0_v, i1_v,
                sr, sa, sb):
    w = lax.axis_index("s") * 2 + lax.axis_index("c")
    cr = pltpu.async_copy(xn_hbm.at[pl.ds(w * TPW, TPW)], rows_v, sr)
    c0 = pltpu.async_copy(s0_hbm.at[pl.ds(w * TPW, TPW)], i0_v, sa)
    c1 = pltpu.async_copy(s1_hbm.at[pl.ds(w * TPW, TPW)], i1_v, sb)
    cr.wait()
    c0.wait()
    c1.wait()
    g0 = pltpu.async_copy(rows_v, xg_hbm.at[i0_v], sa)
    g1 = pltpu.async_copy(rows_v, xg_hbm.at[i1_v], sb)
    g0.wait()
    g1.wait()


_CSET = [
    pltpu.VMEM((CH, D), jnp.float32),
    pltpu.VMEM((CH, D), jnp.float32),
    pltpu.VMEM((CH, D), jnp.float32),
    pltpu.VMEM((CH,), jnp.int32),
    pltpu.VMEM((CH,), jnp.int32),
    pltpu.VMEM((CH, 16), jnp.float32),
    pltpu.VMEM((CH, 16), jnp.float32),
]


@functools.partial(
    pl.kernel, mesh=_SC_MESH,
    out_type=jax.ShapeDtypeStruct((T, D), jnp.float32),
    scratch_types=_CSET + _CSET + [
        pltpu.SemaphoreType.DMA,
        pltpu.SemaphoreType.DMA,
        pltpu.SemaphoreType.DMA,
        pltpu.SemaphoreType.DMA,
    ],
)
def _sc_combine(x_hbm, yg_hbm, s0_hbm, s1_hbm, wq0_hbm, wq1_hbm, out_hbm,
                *bufs):
    sets = (bufs[0:7], bufs[7:14])
    sld = bufs[14:16]
    sg = bufs[16:18]
    w = lax.axis_index("s") * 2 + lax.axis_index("c")
    nch = TPW // CH

    def fire_loads(c):
        xr, _, _, i0, i1, w0, w1 = sets[c % 2]
        sem = sld[c % 2]
        base = w * TPW + c * CH
        return [
            pltpu.async_copy(s0_hbm.at[pl.ds(base, CH)], i0, sem),
            pltpu.async_copy(s1_hbm.at[pl.ds(base, CH)], i1, sem),
            pltpu.async_copy(wq0_hbm.at[pl.ds(base, CH)], w0, sem),
            pltpu.async_copy(wq1_hbm.at[pl.ds(base, CH)], w1, sem),
            pltpu.async_copy(x_hbm.at[pl.ds(base, CH)], xr, sem),
        ]

    def fire_gathers(c):
        _, r0, r1, i0, i1, _, _ = sets[c % 2]
        sem = sg[c % 2]
        return [
            pltpu.async_copy(yg_hbm.at[i0], r0, sem),
            pltpu.async_copy(yg_hbm.at[i1], r1, sem),
        ]

    for h in fire_loads(0):
        h.wait()
    gathers = fire_gathers(0)
    for c in range(nch):
        xr, r0, r1, _, _, w0, w1 = sets[c % 2]
        loads_next = fire_loads(c + 1) if c + 1 < nch else None
        for h in gathers:
            h.wait()
        if loads_next is not None:
            for h in loads_next:
                h.wait()
            gathers = fire_gathers(c + 1)

        def body(r, carry):
            w0s = w0[r, :]
            w1s = w1[r, :]
            for c16 in range(D // 16):
                sl = pl.ds(c16 * 16, 16)
                xr[r, sl] = (xr[r, sl] + r0[r, sl] * w0s
                             + r1[r, sl] * w1s)
            return carry

        lax.fori_loop(0, CH, body, 0)
        pltpu.sync_copy(xr, out_hbm.at[pl.ds(w * TPW + c * CH, CH)])


def _expert_kernel(nblk_ref, blke_ref, xg_ref,
                   w1_ref, b1_ref, w2_ref, b2_ref, yg_ref):
    b = pl.program_id(0)

    @pl.when(b < nblk_ref[0])
    def _():
        h = (jax.lax.dot_general(
            xg_ref[...], w1_ref[0], (((1,), (0,)), ((), ())),
            precision=jax.lax.Precision.DEFAULT,
            preferred_element_type=jnp.float32) + b1_ref[0])
        h = h * jax.nn.sigmoid(h)
        yg_ref[...] = (jax.lax.dot_general(
            h, w2_ref[0], (((1,), (0,)), ((), ())),
            precision=jax.lax.Precision.DEFAULT,
            preferred_element_type=jnp.float32) + b2_ref[0])


def _experts(nblocks, blk_e, xg, w1b, b1r, w2b, b2r):
    def _rowmap(b, n, e):
        return (jnp.minimum(b, n[0] - 1), 0)

    def _emap3(b, n, e):
        return (e[jnp.minimum(b, n[0] - 1)], 0, 0)

    grid_spec = pltpu.PrefetchScalarGridSpec(
        num_scalar_prefetch=2,
        grid=(NB,),
        in_specs=[
            pl.BlockSpec((BG, D), _rowmap),
            pl.BlockSpec((1, D, H), _emap3),
            pl.BlockSpec((1, 1, H), _emap3),
            pl.BlockSpec((1, H, D), _emap3),
            pl.BlockSpec((1, 1, D), _emap3),
        ],
        out_specs=pl.BlockSpec((BG, D), lambda b, n, e: (b, 0)),
    )
    return pl.pallas_call(
        _expert_kernel,
        grid_spec=grid_spec,
        out_shape=jax.ShapeDtypeStruct((GP, D), jnp.float32),
    )(nblocks, blk_e, xg, w1b, b1r, w2b, b2r)


def kernel(x, regime, ln_gamma, ln_beta, W1, b1, W2, b2, Wr1, br1, Wr2, br2):
    x2d = x.reshape(T, D)
    xn, w0, w1, s0, s1, blke2, nblk2, aux = _router(
        x2d, regime, ln_gamma.reshape(1, D), ln_beta.reshape(1, D),
        Wr1, br1.reshape(1, D), Wr2, br2.reshape(1, E))

    s0f = s0.reshape(T)
    s1f = s1.reshape(T)
    xg = _sc_scatter(xn, s0f, s1f)
    yg = _experts(nblk2.reshape(1), blke2.reshape(NB), xg,
                  W1, b1.reshape(E, 1, H),
                  W2, b2.reshape(E, 1, D))
    out2d = _sc_combine(x2d, yg, s0f, s1f, w0, w1)
    return out2d.reshape(B, T, D), aux[0, 0]
